# Initial kernel scaffold; baseline (speedup 1.0000x reference)
import jax
import jax.numpy as jnp
from jax.experimental import pallas as pl


def kernel(x, cluster, edge_index, time_step_len, W0, b0, g0, be0, W1, b1, g1, be1, W2, b2, g2, be2, Wf, bf):
    raise NotImplementedError("write your pallas kernel here")


# speedup vs baseline: 1.3274x; 1.3274x over previous
import functools

import jax
import jax.numpy as jnp
from jax import lax
from jax.experimental import pallas as pl
from jax.experimental.pallas import tpu as pltpu
from jax.experimental.pallas import tpu_sc as plsc

_NC = 2
_NS = 16
_NW = _NC * _NS

_C = 20000

_BLK = 1280
_CH = 400
_ZB = 128
_HP = 128
_LA = 16
_G = 80
_RS = 8


def _sc_mesh():
    return plsc.VectorSubcoreMesh(
        core_axis_name="c", subcore_axis_name="s",
        num_cores=_NC, num_subcores=_NS)


def _ln(h, g, be):
    mu = jnp.mean(h, axis=-1, keepdims=True)
    var = jnp.mean((h - mu) ** 2, axis=-1, keepdims=True)
    return (h - mu) * lax.rsqrt(var + 1e-5) * g + be


def _mlp_first_body(x_ref, w_ref, b_ref, g_ref, be_ref, o_ref):
    h = jnp.dot(x_ref[...], w_ref[...], preferred_element_type=jnp.float32)
    h = h + b_ref[...]
    o_ref[...] = jnp.maximum(_ln(h, g_ref[...], be_ref[...]), 0.0)


def _mlp_pair_body(h_ref, bc_ref, wt_ref, wb_ref, b_ref, g_ref, be_ref, o_ref,
                   *, ln_relu):
    t = jnp.dot(h_ref[...], wt_ref[...], preferred_element_type=jnp.float32)
    t = t + jnp.dot(bc_ref[...], wb_ref[...], preferred_element_type=jnp.float32)
    t = t + b_ref[...]
    if ln_relu:
        t = jnp.maximum(_ln(t, g_ref[...], be_ref[...]), 0.0)
    o_ref[...] = t


def _mlp_first(x, w, b, g, be):
    n, din = x.shape
    h = w.shape[1]
    return pl.pallas_call(
        _mlp_first_body,
        grid=(n // _BLK,),
        in_specs=[
            pl.BlockSpec((_BLK, din), lambda i: (i, 0)),
            pl.BlockSpec((din, h), lambda i: (0, 0)),
            pl.BlockSpec((1, h), lambda i: (0, 0)),
            pl.BlockSpec((1, h), lambda i: (0, 0)),
            pl.BlockSpec((1, h), lambda i: (0, 0)),
        ],
        out_specs=pl.BlockSpec((_BLK, h), lambda i: (i, 0)),
        out_shape=jax.ShapeDtypeStruct((n, h), jnp.float32),
    )(x, w, b.reshape(1, -1), g.reshape(1, -1), be.reshape(1, -1))


def _mlp_pair(hin, bc, wt, wb, b, g, be, ln_relu):
    n, h = hin.shape
    bw = bc.shape[1]
    return pl.pallas_call(
        functools.partial(_mlp_pair_body, ln_relu=ln_relu),
        grid=(n // _BLK,),
        in_specs=[
            pl.BlockSpec((_BLK, h), lambda i: (i, 0)),
            pl.BlockSpec((_BLK, bw), lambda i: (i, 0)),
            pl.BlockSpec((h, h), lambda i: (0, 0)),
            pl.BlockSpec((bw, h), lambda i: (0, 0)),
            pl.BlockSpec((1, h), lambda i: (0, 0)),
            pl.BlockSpec((1, h), lambda i: (0, 0)),
            pl.BlockSpec((1, h), lambda i: (0, 0)),
        ],
        out_specs=pl.BlockSpec((_BLK, h), lambda i: (i, 0)),
        out_shape=jax.ShapeDtypeStruct((n, h), jnp.float32),
    )(hin, bc, wt, wb, b.reshape(1, -1), g.reshape(1, -1), be.reshape(1, -1))


def _normalize(a, c_out, hd_out):
    def body(a_ref, o_ref):
        v = a_ref[:c_out, :hd_out]
        nrm = jnp.sqrt(jnp.sum(v * v, axis=-1, keepdims=True))
        o_ref[...] = v / jnp.maximum(nrm, 1e-12)

    return pl.pallas_call(
        body, out_shape=jax.ShapeDtypeStruct((c_out, hd_out), jnp.float32))(a)


def _seg_max(hmat, cl):
    n, hd = hmat.shape
    r_per = n // _NW
    nk = hd // 16
    npk = _HP // 16
    nch = r_per // _CH

    @functools.partial(
        pl.kernel,
        out_type=jax.ShapeDtypeStruct((_C * _HP,), jnp.float32),
        mesh=_sc_mesh(),
        scratch_types=[
            pltpu.VMEM((2 * _CH,), jnp.int32),
            pltpu.VMEM((2 * _CH * hd,), jnp.float32),
            pltpu.VMEM((_RS * _HP,), jnp.float32),
            pltpu.VMEM((_ZB * _HP,), jnp.float32),
            pltpu.VMEM((16,), jnp.int32),
            pltpu.VMEM((16,), jnp.int32),
            pltpu.VMEM((64,), jnp.int32),
            pltpu.VMEM((_LA * hd,), jnp.float32),
            pltpu.SemaphoreType.DMA,
            pltpu.SemaphoreType.DMA,
        ],
    )
    def k(h_hbm, cl_hbm, agg_hbm, idv, hv, accr, zv, bidv, bidv2, laidv, lahv,
          semc, semf):
        wid = lax.axis_index("s") * _NC + lax.axis_index("c")
        row0 = wid * r_per

        pltpu.sync_copy(
            cl_hbm.at[pl.ds(pl.multiple_of(jnp.maximum(row0 - 16, 0), 16), 16)],
            bidv)
        prev_id = jnp.where(wid > 0, bidv[...][15], -1)
        nxt_base = jnp.minimum(row0 + r_per, n - 16)
        pltpu.sync_copy(
            cl_hbm.at[pl.ds(pl.multiple_of(nxt_base, 16), 16)], bidv2)
        a_hi = jnp.where(wid < _NW - 1, bidv2[...][0], _C)
        pltpu.sync_copy(cl_hbm.at[pl.ds(pl.multiple_of(row0, 16), 16)], bidv)
        first_id = bidv[...][0]

        pltpu.async_copy(
            cl_hbm.at[pl.ds(pl.multiple_of(row0, 16), _CH)],
            idv.at[0:_CH], semc)
        pltpu.async_copy(
            h_hbm.at[pl.ds(pl.multiple_of(row0 * hd, 8), _CH * hd)],
            hv.at[0:_CH * hd], semc)

        def zfill(j, _):
            zv[pl.ds(j * 16, 16)] = jnp.zeros((16,), jnp.float32)
            return 0
        lax.fori_loop(0, (_ZB * _HP) // 16, zfill, 0)
        for s in range(_RS):
            for kk in range(nk, npk):
                accr[pl.ds(s * _HP + kk * 16, 16)] = jnp.zeros(
                    (16,), jnp.float32)

        z0 = jnp.where(wid == 0, 0,
                       first_id + jnp.where(prev_id == first_id, 1, 0))
        zcnt = jnp.maximum(a_hi - z0, 0)
        pos = z0
        for sz in (_ZB, 16, 1):
            def zbody(i, p, sz=sz):
                pltpu.sync_copy(
                    zv.at[0:sz * _HP],
                    agg_hbm.at[pl.ds(pl.multiple_of(p * _HP, 8), sz * _HP)])
                return p + sz
            cnt = zcnt // sz
            pos = lax.fori_loop(0, cnt, zbody, pos)
            zcnt = zcnt - cnt * sz

        def fire(cur_id, aoff):
            pltpu.async_copy(
                accr.at[pl.ds(aoff, _HP)],
                agg_hbm.at[pl.ds(pl.multiple_of(cur_id * _HP, 8), _HP)],
                semf)

        def drain1():
            pltpu.make_async_copy(
                accr.at[pl.ds(0, _HP)], agg_hbm.at[pl.ds(0, _HP)], semf).wait()

        def make_group_body(p_off_id, p_off_h):
            def group_body(g, carry):
                cur_id, own, slot, pend = carry
                idvec = idv[pl.ds(p_off_id + g * 16, 16)]
                for j in range(16):
                    rid = idvec[j]
                    same = rid == cur_id
                    i = g * 16 + j
                    fin = jnp.logical_and(jnp.logical_not(same), own == 1)
                    aoff = pl.multiple_of(slot * _HP, 8)

                    @pl.when(jnp.logical_and(same, own == 1))
                    def _(i=i, aoff=aoff):
                        for kk in range(nk):
                            accr[pl.ds(aoff + kk * 16, 16)] = jnp.maximum(
                                accr[pl.ds(aoff + kk * 16, 16)],
                                hv[pl.ds(p_off_h + i * hd + kk * 16, 16)])

                    @pl.when(fin)
                    def _(cur_id=cur_id, aoff=aoff):
                        fire(cur_id, aoff)

                    slot = jnp.where(fin, (slot + 1) % _RS, slot)
                    pend = jnp.where(fin, pend + 1, pend)
                    do_drain = pend > _RS - 1

                    @pl.when(do_drain)
                    def _():
                        drain1()

                    pend = jnp.where(do_drain, pend - 1, pend)
                    aoff2 = pl.multiple_of(slot * _HP, 8)

                    @pl.when(jnp.logical_not(same))
                    def _(i=i, aoff2=aoff2):
                        for kk in range(nk):
                            accr[pl.ds(aoff2 + kk * 16, 16)] = hv[
                                pl.ds(p_off_h + i * hd + kk * 16, 16)]

                    cur_id = jnp.where(same, cur_id, rid)
                    own = jnp.where(same, own, 1)
                return (cur_id, own, slot, pend)
            return group_body

        def chunk_body(ch, carry):
            p = ch % 2
            p_off_id = pl.multiple_of(p * _CH, 16)
            p_off_h = pl.multiple_of(p * _CH * hd, 8)
            pltpu.make_async_copy(
                cl_hbm.at[pl.ds(0, _CH)], idv.at[0:_CH], semc).wait()
            pltpu.make_async_copy(
                h_hbm.at[pl.ds(0, _CH * hd)], hv.at[0:_CH * hd], semc).wait()

            @pl.when(ch + 1 < nch)
            def _():
                q = (ch + 1) % 2
                base2 = row0 + (ch + 1) * _CH
                pltpu.async_copy(
                    cl_hbm.at[pl.ds(pl.multiple_of(base2, 16), _CH)],
                    idv.at[pl.ds(pl.multiple_of(q * _CH, 16), _CH)], semc)
                pltpu.async_copy(
                    h_hbm.at[pl.ds(pl.multiple_of(base2 * hd, 8), _CH * hd)],
                    hv.at[pl.ds(pl.multiple_of(q * _CH * hd, 8), _CH * hd)],
                    semc)

            return lax.fori_loop(
                0, _CH // 16, make_group_body(p_off_id, p_off_h), carry)

        cur_id, own, slot, pend = lax.fori_loop(
            0, nch, chunk_body,
            (prev_id, jnp.int32(0), jnp.int32(0), jnp.int32(0)))

        rend = row0 + r_per
        probe_base = jnp.minimum(rend, n - 64)
        pltpu.sync_copy(
            cl_hbm.at[pl.ds(pl.multiple_of(probe_base, 16), 64)], laidv)
        la_cnt = jnp.int32(0)
        still = own == 1
        for kg in range(4):
            valid = rend + kg * 16 + 16 <= n
            eqi = jnp.where(laidv[pl.ds(kg * 16, 16)] == cur_id, 1, 0)
            pc = jnp.int32(0)
            ok = jnp.int32(1)
            for j in range(16):
                ok = ok * eqi[j]
                pc = pc + ok
            take = jnp.logical_and(still, valid)
            la_cnt = jnp.where(take, la_cnt + pc, la_cnt)
            still = jnp.logical_and(
                take, jnp.logical_and(pc == 16, rend + kg * 16 + 16 < n))

        trip_a = jnp.where(still, (n - rend - 64) // 16, 0)

        def ext_body(i, st):
            cnt, stl = st
            base = rend + 64 + i * 16
            pltpu.sync_copy(
                cl_hbm.at[pl.ds(pl.multiple_of(base, 16), 16)], bidv)
            eqi = jnp.where(bidv[...] == cur_id, 1, 0)
            pc = jnp.int32(0)
            ok = jnp.int32(1)
            for j in range(16):
                ok = ok * eqi[j]
                pc = pc + ok
            cnt = jnp.where(stl, cnt + pc, cnt)
            stl = jnp.logical_and(stl, jnp.logical_and(pc == 16, base + 16 < n))
            return (cnt, stl)

        la_cnt, _ = lax.fori_loop(0, trip_a, ext_body, (la_cnt, still))

        aofff = pl.multiple_of(slot * _HP, 8)

        def lb_body(i, _):
            base = rend + i * 16
            rem = la_cnt - i * 16
            pltpu.sync_copy(
                h_hbm.at[pl.ds(pl.multiple_of(base * hd, 8), _LA * hd)], lahv)
            for j in range(16):
                @pl.when(j < rem)
                def _(j=j):
                    for kk in range(nk):
                        accr[pl.ds(aofff + kk * 16, 16)] = jnp.maximum(
                            accr[pl.ds(aofff + kk * 16, 16)],
                            lahv[pl.ds(j * hd + kk * 16, 16)])
            return 0

        lax.fori_loop(0, (la_cnt + 15) // 16, lb_body, 0)

        @pl.when(own == 1)
        def _():
            fire(cur_id, aofff)

        pend = jnp.where(own == 1, pend + 1, pend)

        def drain_body(i, _):
            drain1()
            return 0

        lax.fori_loop(0, pend, drain_body, 0)

    return k(hmat.reshape(-1), cl).reshape(_C, _HP)


def _gather_rows(agg, cl, n):
    c_total, hd = agg.shape
    r_per = n // _NW
    nchg = r_per // _G

    @functools.partial(
        pl.kernel,
        out_type=jax.ShapeDtypeStruct((n, hd), jnp.float32),
        mesh=_sc_mesh(),
        scratch_types=[
            pltpu.VMEM((r_per,), jnp.int32),
            pltpu.VMEM((2, _G, hd), jnp.float32),
            pltpu.SemaphoreType.DMA,
            pltpu.SemaphoreType.DMA,
        ],
    )
    def k(agg_hbm, cl_hbm, out_hbm, idxall, rows3, semg, semo):
        wid = lax.axis_index("s") * _NC + lax.axis_index("c")
        base0 = wid * r_per
        pltpu.sync_copy(
            cl_hbm.at[pl.ds(pl.multiple_of(base0, 16), r_per)], idxall)

        pltpu.async_copy(
            agg_hbm.at[idxall.at[pl.ds(0, _G)]], rows3.at[0], semg)

        def body(ch, _):
            p = ch % 2
            pltpu.make_async_copy(
                agg_hbm.at[idxall.at[pl.ds(0, _G)]], rows3.at[0], semg).wait()

            @pl.when(ch >= 1)
            def _():
                pltpu.make_async_copy(
                    rows3.at[0], out_hbm.at[pl.ds(0, _G)], semo).wait()

            @pl.when(ch + 1 < nchg)
            def _():
                q = (ch + 1) % 2
                pltpu.async_copy(
                    agg_hbm.at[idxall.at[pl.ds(
                        pl.multiple_of((ch + 1) * _G, 16), _G)]],
                    rows3.at[q], semg)

            pltpu.async_copy(
                rows3.at[p],
                out_hbm.at[pl.ds(pl.multiple_of(base0 + ch * _G, 16), _G)],
                semo)
            return 0

        lax.fori_loop(0, nchg, body, 0)
        pltpu.make_async_copy(
            rows3.at[0], out_hbm.at[pl.ds(0, _G)], semo).wait()

    return k(agg, cl)


def kernel(x, cluster, edge_index, time_step_len,
           W0, b0, g0, be0, W1, b1, g1, be1, W2, b2, g2, be2, Wf, bf):
    n = x.shape[0]
    hd = W0.shape[1]

    def pad_bot(w):
        return jnp.zeros((_HP, w.shape[1]), w.dtype).at[:w.shape[0]].set(w)

    h = _mlp_first(x, W0, b0, g0, be0)
    for (W, b, g, be) in ((W1, b1, g1, be1), (W2, b2, g2, be2)):
        agg = _seg_max(h, cluster)
        bc = _gather_rows(agg, cluster, n)
        h = _mlp_pair(h, bc, W[:hd], pad_bot(W[hd:]), b, g, be, ln_relu=True)
    agg = _seg_max(h, cluster)
    bc = _gather_rows(agg, cluster, n)
    hf = _mlp_pair(h, bc, Wf[:hd], pad_bot(Wf[hd:]), bf, bf, bf, ln_relu=False)
    aggf = _seg_max(hf, cluster)
    return _normalize(aggf, _C, hd)
